# Initial kernel scaffold; baseline (speedup 1.0000x reference)
import jax
import jax.numpy as jnp
from jax.experimental import pallas as pl


def kernel(s_j, v_j, r_ij, nbrs, cg_adj, W1, b1, W2, b2, Wd, bd):
    raise NotImplementedError("write your pallas kernel here")


# speedup vs baseline: 4.5194x; 4.5194x over previous
import functools

import jax
import jax.numpy as jnp
from jax.experimental import pallas as pl

N = 10000
E = 320000
FEAT = 128
NRBF = 20
CUTOFF = 5.0
EPS = 1e-15

BLK_E = 1600


def _edge_block_kernel(r_ref, phi_ref, vdx_ref, vdy_ref, vdz_ref,
                       vsx_ref, vsy_ref, vsz_ref, adj_ref, wd_ref, bd_ref,
                       coef_ref,
                       ds_ref, dvx_ref, dvy_ref, dvz_ref):
    r = r_ref[...]
    rx = r[:, 0:1]
    ry = r[:, 1:2]
    rz = r[:, 2:3]
    d2 = rx * rx + ry * ry + rz * rz + EPS
    dist = jnp.sqrt(d2)
    inv_d = 1.0 / dist
    ux, uy, uz = rx * inv_d, ry * inv_d, rz * inv_d

    rbf = jnp.sin(coef_ref[...] * dist) * inv_d
    rbf_feats = jnp.dot(rbf, wd_ref[...],
                        preferred_element_type=jnp.float32) + bd_ref[...]
    env = jnp.where(dist < CUTOFF,
                    0.5 * (jnp.cos((jnp.pi / CUTOFF) * dist) + 1.0), 0.0)
    w_s = rbf_feats * env

    phi = phi_ref[...]
    inv_out = phi * w_s
    adj = adj_ref[...]
    s0 = inv_out[:, 0 * FEAT:1 * FEAT] * adj
    s1 = inv_out[:, 1 * FEAT:2 * FEAT] * adj
    s2 = inv_out[:, 2 * FEAT:3 * FEAT] * adj
    s3 = inv_out[:, 3 * FEAT:4 * FEAT] * adj

    vdx, vdy, vdz = vdx_ref[...], vdy_ref[...], vdz_ref[...]
    vsx, vsy, vsz = vsx_ref[...], vsy_ref[...], vsz_ref[...]
    cx = vsy * vdz - vsz * vdy
    cy = vsz * vdx - vsx * vdz
    cz = vsx * vdy - vsy * vdx

    ds_ref[...] = s1
    dvx_ref[...] = s2 * ux + s0 * vdx + s3 * cx
    dvy_ref[...] = s2 * uy + s0 * vdy + s3 * cy
    dvz_ref[...] = s2 * uz + s0 * vdz + s3 * cz


def _edge_pass(r_ij, phi_e, vd, vs, adj, Wd4, bd4, interpret=False):
    grid = (E // BLK_E,)
    eb = lambda w: pl.BlockSpec((BLK_E, w), lambda i: (i, 0))
    full = lambda a, b: pl.BlockSpec((a, b), lambda i: (0, 0))
    out_shapes = (
        jax.ShapeDtypeStruct((E, FEAT), jnp.float32),
        jax.ShapeDtypeStruct((E, FEAT), jnp.float32),
        jax.ShapeDtypeStruct((E, FEAT), jnp.float32),
        jax.ShapeDtypeStruct((E, FEAT), jnp.float32),
    )
    coef = (jnp.arange(1, NRBF + 1, dtype=jnp.float32)
            * (jnp.pi / CUTOFF))[None, :]
    return pl.pallas_call(
        _edge_block_kernel,
        grid=grid,
        in_specs=[eb(3), eb(4 * FEAT),
                  eb(FEAT), eb(FEAT), eb(FEAT),
                  eb(FEAT), eb(FEAT), eb(FEAT),
                  eb(1), full(NRBF, 4 * FEAT), full(1, 4 * FEAT),
                  full(1, NRBF)],
        out_specs=(eb(FEAT), eb(FEAT), eb(FEAT), eb(FEAT)),
        out_shape=out_shapes,
        interpret=interpret,
    )(r_ij, phi_e, vd[:, 0], vd[:, 1], vd[:, 2],
      vs[:, 0], vs[:, 1], vs[:, 2], adj, Wd4, bd4, coef)


@jax.jit
def kernel(s_j, v_j, r_ij, nbrs, cg_adj, W1, b1, W2, b2, Wd, bd):
    h = s_j @ W1 + b1
    h = h * jax.nn.sigmoid(h)
    phi_all = h @ W2 + b2

    src = nbrs[:, 0]
    dst = nbrs[:, 1]
    phi_e = phi_all[dst]
    vt = jnp.swapaxes(v_j, 1, 2)
    vd = vt[dst]
    vs = vt[src]
    adj = cg_adj[src, dst][:, None]

    ds_ij, dvx, dvy, dvz = _edge_pass(r_ij, phi_e, vd, vs, adj,
                                      Wd, bd[None, :])

    ds_i = jnp.zeros((N, FEAT), jnp.float32).at[src].add(ds_ij)
    dvx_i = jnp.zeros((N, FEAT), jnp.float32).at[src].add(dvx)
    dvy_i = jnp.zeros((N, FEAT), jnp.float32).at[src].add(dvy)
    dvz_i = jnp.zeros((N, FEAT), jnp.float32).at[src].add(dvz)
    dv_i = jnp.stack([dvx_i, dvy_i, dvz_i], axis=-1)
    return ds_i, dv_i


# speedup vs baseline: 4.6390x; 1.0265x over previous
import functools

import jax
import jax.numpy as jnp
from jax.experimental import pallas as pl

N = 10000
E = 320000
FEAT = 128
NRBF = 20
CUTOFF = 5.0
EPS = 1e-15

BLK_E = 1600


def _edge_block_kernel(r_ref, phi_ref, vdx_ref, vdy_ref, vdz_ref,
                       vsx_ref, vsy_ref, vsz_ref, adj_ref, wd_ref, bd_ref,
                       coef_ref,
                       ds_ref, dvx_ref, dvy_ref, dvz_ref):
    r = r_ref[...]
    rx = r[:, 0:1]
    ry = r[:, 1:2]
    rz = r[:, 2:3]
    d2 = rx * rx + ry * ry + rz * rz + EPS
    dist = jnp.sqrt(d2)
    inv_d = 1.0 / dist
    ux, uy, uz = rx * inv_d, ry * inv_d, rz * inv_d

    rbf = jnp.sin(coef_ref[...] * dist) * inv_d
    rbf_feats = jnp.dot(rbf, wd_ref[...],
                        preferred_element_type=jnp.float32) + bd_ref[...]
    env = jnp.where(dist < CUTOFF,
                    0.5 * (jnp.cos((jnp.pi / CUTOFF) * dist) + 1.0), 0.0)
    w_s = rbf_feats * env

    phi = phi_ref[...]
    inv_out = phi * w_s
    adj = adj_ref[...]
    s0 = inv_out[:, 0 * FEAT:1 * FEAT] * adj
    s1 = inv_out[:, 1 * FEAT:2 * FEAT] * adj
    s2 = inv_out[:, 2 * FEAT:3 * FEAT] * adj
    s3 = inv_out[:, 3 * FEAT:4 * FEAT] * adj

    vdx, vdy, vdz = vdx_ref[...], vdy_ref[...], vdz_ref[...]
    vsx, vsy, vsz = vsx_ref[...], vsy_ref[...], vsz_ref[...]
    cx = vsy * vdz - vsz * vdy
    cy = vsz * vdx - vsx * vdz
    cz = vsx * vdy - vsy * vdx

    ds_ref[...] = s1
    dvx_ref[...] = s2 * ux + s0 * vdx + s3 * cx
    dvy_ref[...] = s2 * uy + s0 * vdy + s3 * cy
    dvz_ref[...] = s2 * uz + s0 * vdz + s3 * cz


def _edge_pass(r_ij, phi_e, vd, vs, adj, Wd4, bd4, interpret=False):
    grid = (E // BLK_E,)
    eb = lambda w: pl.BlockSpec((BLK_E, w), lambda i: (i, 0))
    full = lambda a, b: pl.BlockSpec((a, b), lambda i: (0, 0))
    out_shapes = (
        jax.ShapeDtypeStruct((E, FEAT), jnp.float32),
        jax.ShapeDtypeStruct((E, FEAT), jnp.float32),
        jax.ShapeDtypeStruct((E, FEAT), jnp.float32),
        jax.ShapeDtypeStruct((E, FEAT), jnp.float32),
    )
    coef = (jnp.arange(1, NRBF + 1, dtype=jnp.float32)
            * (jnp.pi / CUTOFF))[None, :]
    return pl.pallas_call(
        _edge_block_kernel,
        grid=grid,
        in_specs=[eb(3), eb(4 * FEAT),
                  eb(FEAT), eb(FEAT), eb(FEAT),
                  eb(FEAT), eb(FEAT), eb(FEAT),
                  eb(1), full(NRBF, 4 * FEAT), full(1, 4 * FEAT),
                  full(1, NRBF)],
        out_specs=(eb(FEAT), eb(FEAT), eb(FEAT), eb(FEAT)),
        out_shape=out_shapes,
        interpret=interpret,
    )(r_ij, phi_e, vd[:, 0], vd[:, 1], vd[:, 2],
      vs[:, 0], vs[:, 1], vs[:, 2], adj, Wd4, bd4, coef)


@jax.jit
def kernel(s_j, v_j, r_ij, nbrs, cg_adj, W1, b1, W2, b2, Wd, bd):
    h = s_j @ W1 + b1
    h = h * jax.nn.sigmoid(h)
    phi_all = h @ W2 + b2

    src = nbrs[:, 0]
    dst = nbrs[:, 1]
    vt = jnp.swapaxes(v_j, 1, 2)
    phi_e = phi_all[dst]
    vd = vt[dst]
    vs = vt[src]
    adj = cg_adj[src, dst][:, None]

    ds_ij, dvx, dvy, dvz = _edge_pass(r_ij, phi_e, vd, vs, adj,
                                      Wd, bd[None, :])

    msgs = jnp.concatenate([ds_ij, dvx, dvy, dvz], axis=1)
    acc = jnp.zeros((N, 4 * FEAT), jnp.float32).at[src].add(msgs)
    ds_i = acc[:, :FEAT]
    dv_i = jnp.stack([acc[:, FEAT:2 * FEAT], acc[:, 2 * FEAT:3 * FEAT],
                      acc[:, 3 * FEAT:]], axis=-1)
    return ds_i, dv_i


# speedup vs baseline: 5.1702x; 1.1145x over previous
import functools

import jax
import jax.numpy as jnp
from jax import lax
from jax.experimental import pallas as pl
from jax.experimental.pallas import tpu as pltpu
from jax.experimental.pallas import tpu_sc as plsc

N = 10000
E = 320000
FEAT = 128
NRBF = 20
CUTOFF = 5.0
EPS = 1e-15

BLK_E = 1600


def _edge_block_kernel(r_ref, phi_ref, vdx_ref, vdy_ref, vdz_ref,
                       vsx_ref, vsy_ref, vsz_ref, adj_ref, wd_ref, bd_ref,
                       coef_ref,
                       ds_ref, dvx_ref, dvy_ref, dvz_ref):
    r = r_ref[...]
    rx = r[:, 0:1]
    ry = r[:, 1:2]
    rz = r[:, 2:3]
    d2 = rx * rx + ry * ry + rz * rz + EPS
    dist = jnp.sqrt(d2)
    inv_d = 1.0 / dist
    ux, uy, uz = rx * inv_d, ry * inv_d, rz * inv_d

    rbf = jnp.sin(coef_ref[...] * dist) * inv_d
    rbf_feats = jnp.dot(rbf, wd_ref[...],
                        preferred_element_type=jnp.float32) + bd_ref[...]
    env = jnp.where(dist < CUTOFF,
                    0.5 * (jnp.cos((jnp.pi / CUTOFF) * dist) + 1.0), 0.0)
    w_s = rbf_feats * env

    phi = phi_ref[...]
    inv_out = phi * w_s
    adj = adj_ref[...]
    s0 = inv_out[:, 0 * FEAT:1 * FEAT] * adj
    s1 = inv_out[:, 1 * FEAT:2 * FEAT] * adj
    s2 = inv_out[:, 2 * FEAT:3 * FEAT] * adj
    s3 = inv_out[:, 3 * FEAT:4 * FEAT] * adj

    vdx, vdy, vdz = vdx_ref[...], vdy_ref[...], vdz_ref[...]
    vsx, vsy, vsz = vsx_ref[...], vsy_ref[...], vsz_ref[...]
    cx = vsy * vdz - vsz * vdy
    cy = vsz * vdx - vsx * vdz
    cz = vsx * vdy - vsy * vdx

    ds_ref[...] = s1
    dvx_ref[...] = s2 * ux + s0 * vdx + s3 * cx
    dvy_ref[...] = s2 * uy + s0 * vdy + s3 * cy
    dvz_ref[...] = s2 * uz + s0 * vdz + s3 * cz


def _edge_pass(r_ij, phi_e, vd, vs, adj, Wd4, bd4, interpret=False):
    grid = (E // BLK_E,)
    eb = lambda w: pl.BlockSpec((BLK_E, w), lambda i: (i, 0))
    full = lambda a, b: pl.BlockSpec((a, b), lambda i: (0, 0))
    out_shapes = (
        jax.ShapeDtypeStruct((E, FEAT), jnp.float32),
        jax.ShapeDtypeStruct((E, FEAT), jnp.float32),
        jax.ShapeDtypeStruct((E, FEAT), jnp.float32),
        jax.ShapeDtypeStruct((E, FEAT), jnp.float32),
    )
    coef = (jnp.arange(1, NRBF + 1, dtype=jnp.float32)
            * (jnp.pi / CUTOFF))[None, :]
    return pl.pallas_call(
        _edge_block_kernel,
        grid=grid,
        in_specs=[eb(3), eb(4 * FEAT),
                  eb(FEAT), eb(FEAT), eb(FEAT),
                  eb(FEAT), eb(FEAT), eb(FEAT),
                  eb(1), full(NRBF, 4 * FEAT), full(1, 4 * FEAT),
                  full(1, NRBF)],
        out_specs=(eb(FEAT), eb(FEAT), eb(FEAT), eb(FEAT)),
        out_shape=out_shapes,
        interpret=interpret,
    )(r_ij, phi_e, vd[:, 0], vd[:, 1], vd[:, 2],
      vs[:, 0], vs[:, 1], vs[:, 2], adj, Wd4, bd4, coef)


_NC, _NS = 2, 16
_NW = _NC * _NS
_EPW = E // _NW
_GB = 80


def _sc_gather_body(phi_hbm, vt_hbm, dst_hbm, src_hbm,
                    phi_out, vd_out, vs_out,
                    idx_d, idx_s, phi_rows, vd_rows, vs_rows,
                    sem_p, sem_d, sem_s):
    wid = lax.axis_index("s") * _NC + lax.axis_index("c")
    wbase = wid * _EPW

    def chunk(i, _):
        base = wbase + i * _GB
        pltpu.sync_copy(dst_hbm.at[pl.ds(base, _GB)], idx_d)
        pltpu.sync_copy(src_hbm.at[pl.ds(base, _GB)], idx_s)
        cp_p = pltpu.async_copy(phi_hbm.at[idx_d], phi_rows, sem_p)
        cp_d = pltpu.async_copy(vt_hbm.at[idx_d], vd_rows, sem_d)
        cp_s = pltpu.async_copy(vt_hbm.at[idx_s], vs_rows, sem_s)
        cp_p.wait()
        pltpu.sync_copy(phi_rows, phi_out.at[pl.ds(base, _GB)])
        cp_d.wait()
        pltpu.sync_copy(vd_rows, vd_out.at[pl.ds(base, _GB)])
        cp_s.wait()
        pltpu.sync_copy(vs_rows, vs_out.at[pl.ds(base, _GB)])
        return _

    lax.fori_loop(0, _EPW // _GB, chunk, 0)


def _sc_gather(phi_all, vt2d, dst_i32, src_i32):
    mesh = plsc.VectorSubcoreMesh(core_axis_name="c", subcore_axis_name="s")
    f = pl.kernel(
        _sc_gather_body,
        mesh=mesh,
        out_type=(jax.ShapeDtypeStruct((E, 4 * FEAT), jnp.float32),
                  jax.ShapeDtypeStruct((E, 3 * FEAT), jnp.float32),
                  jax.ShapeDtypeStruct((E, 3 * FEAT), jnp.float32)),
        scratch_types=[
            pltpu.VMEM((_GB,), jnp.int32),
            pltpu.VMEM((_GB,), jnp.int32),
            pltpu.VMEM((_GB, 4 * FEAT), jnp.float32),
            pltpu.VMEM((_GB, 3 * FEAT), jnp.float32),
            pltpu.VMEM((_GB, 3 * FEAT), jnp.float32),
            pltpu.SemaphoreType.DMA,
            pltpu.SemaphoreType.DMA,
            pltpu.SemaphoreType.DMA,
        ],
    )
    return f(phi_all, vt2d, dst_i32, src_i32)


@jax.jit
def kernel(s_j, v_j, r_ij, nbrs, cg_adj, W1, b1, W2, b2, Wd, bd):
    h = s_j @ W1 + b1
    h = h * jax.nn.sigmoid(h)
    phi_all = h @ W2 + b2

    src = nbrs[:, 0]
    dst = nbrs[:, 1]
    vt = jnp.swapaxes(v_j, 1, 2)
    vt2d = vt.reshape(N, 3 * FEAT)
    phi_e, vde, vse = _sc_gather(phi_all, vt2d, dst.astype(jnp.int32),
                                 src.astype(jnp.int32))
    vd = vde.reshape(E, 3, FEAT)
    vs = vse.reshape(E, 3, FEAT)
    adj = cg_adj[src, dst][:, None]

    ds_ij, dvx, dvy, dvz = _edge_pass(r_ij, phi_e, vd, vs, adj,
                                      Wd, bd[None, :])

    msgs = jnp.concatenate([ds_ij, dvx, dvy, dvz], axis=1)
    acc = jnp.zeros((N, 4 * FEAT), jnp.float32).at[src].add(msgs)
    ds_i = acc[:, :FEAT]
    dv_i = jnp.stack([acc[:, FEAT:2 * FEAT], acc[:, 2 * FEAT:3 * FEAT],
                      acc[:, 3 * FEAT:]], axis=-1)
    return ds_i, dv_i


# speedup vs baseline: 7.7989x; 1.5084x over previous
import functools

import jax
import jax.numpy as jnp
from jax import lax
from jax.experimental import pallas as pl
from jax.experimental.pallas import tpu as pltpu
from jax.experimental.pallas import tpu_sc as plsc

N = 10000
E = 320000
FEAT = 128
NRBF = 20
CUTOFF = 5.0
EPS = 1e-15

BLK_E = 1600


def _edge_block_kernel(r_ref, phi_ref, vd_ref, vs_ref, adj_ref, wd_ref,
                       bd_ref, coef_ref, ms_ref):
    r = r_ref[...]
    rx = r[:, 0:1]
    ry = r[:, 1:2]
    rz = r[:, 2:3]
    d2 = rx * rx + ry * ry + rz * rz + EPS
    dist = jnp.sqrt(d2)
    inv_d = 1.0 / dist
    ux, uy, uz = rx * inv_d, ry * inv_d, rz * inv_d

    rbf = jnp.sin(coef_ref[...] * dist) * inv_d
    rbf_feats = jnp.dot(rbf, wd_ref[...],
                        preferred_element_type=jnp.float32) + bd_ref[...]
    env = jnp.where(dist < CUTOFF,
                    0.5 * (jnp.cos((jnp.pi / CUTOFF) * dist) + 1.0), 0.0)
    w_s = rbf_feats * env

    phi = phi_ref[...]
    inv_out = phi * w_s
    adj = adj_ref[...]
    s0 = inv_out[:, 0 * FEAT:1 * FEAT] * adj
    s1 = inv_out[:, 1 * FEAT:2 * FEAT] * adj
    s2 = inv_out[:, 2 * FEAT:3 * FEAT] * adj
    s3 = inv_out[:, 3 * FEAT:4 * FEAT] * adj

    vd = vd_ref[...]
    vs = vs_ref[...]
    vdx, vdy, vdz = (vd[:, 0:FEAT], vd[:, FEAT:2 * FEAT],
                     vd[:, 2 * FEAT:3 * FEAT])
    vsx, vsy, vsz = (vs[:, 0:FEAT], vs[:, FEAT:2 * FEAT],
                     vs[:, 2 * FEAT:3 * FEAT])
    cx = vsy * vdz - vsz * vdy
    cy = vsz * vdx - vsx * vdz
    cz = vsx * vdy - vsy * vdx

    ms_ref[:, 0 * FEAT:1 * FEAT] = s1
    ms_ref[:, 1 * FEAT:2 * FEAT] = s2 * ux + s0 * vdx + s3 * cx
    ms_ref[:, 2 * FEAT:3 * FEAT] = s2 * uy + s0 * vdy + s3 * cy
    ms_ref[:, 3 * FEAT:4 * FEAT] = s2 * uz + s0 * vdz + s3 * cz


def _edge_pass(r_ij, phi_e, vde, vse, adj, Wd4, bd4, interpret=False):
    grid = (E // BLK_E,)
    eb = lambda w: pl.BlockSpec((BLK_E, w), lambda i: (i, 0))
    full = lambda a, b: pl.BlockSpec((a, b), lambda i: (0, 0))
    coef = (jnp.arange(1, NRBF + 1, dtype=jnp.float32)
            * (jnp.pi / CUTOFF))[None, :]
    return pl.pallas_call(
        _edge_block_kernel,
        grid=grid,
        in_specs=[eb(3), eb(4 * FEAT), eb(3 * FEAT), eb(3 * FEAT),
                  eb(1), full(NRBF, 4 * FEAT), full(1, 4 * FEAT),
                  full(1, NRBF)],
        out_specs=eb(4 * FEAT),
        out_shape=jax.ShapeDtypeStruct((E, 4 * FEAT), jnp.float32),
        interpret=interpret,
    )(r_ij, phi_e, vde, vse, adj, Wd4, bd4, coef)


_NC, _NS = 2, 16
_NW = _NC * _NS
_EPW = E // _NW
_GB = 80


def _sc_gather_body(phi_hbm, vt_hbm, dst_hbm, src_hbm,
                    phi_out, vd_out, vs_out,
                    idx_d, idx_s, phi_rows, vd_rows, vs_rows,
                    sem_p, sem_d, sem_s):
    wid = lax.axis_index("s") * _NC + lax.axis_index("c")
    wbase = wid * _EPW

    def chunk(i, _):
        base = wbase + i * _GB
        pltpu.sync_copy(dst_hbm.at[pl.ds(base, _GB)], idx_d)
        pltpu.sync_copy(src_hbm.at[pl.ds(base, _GB)], idx_s)
        cp_p = pltpu.async_copy(phi_hbm.at[idx_d], phi_rows, sem_p)
        cp_d = pltpu.async_copy(vt_hbm.at[idx_d], vd_rows, sem_d)
        cp_s = pltpu.async_copy(vt_hbm.at[idx_s], vs_rows, sem_s)
        cp_p.wait()
        pltpu.sync_copy(phi_rows, phi_out.at[pl.ds(base, _GB)])
        cp_d.wait()
        pltpu.sync_copy(vd_rows, vd_out.at[pl.ds(base, _GB)])
        cp_s.wait()
        pltpu.sync_copy(vs_rows, vs_out.at[pl.ds(base, _GB)])
        return _

    lax.fori_loop(0, _EPW // _GB, chunk, 0)


def _sc_gather(phi_all, vt2d, dst_i32, src_i32):
    mesh = plsc.VectorSubcoreMesh(core_axis_name="c", subcore_axis_name="s")
    f = pl.kernel(
        _sc_gather_body,
        mesh=mesh,
        out_type=(jax.ShapeDtypeStruct((E, 4 * FEAT), jnp.float32),
                  jax.ShapeDtypeStruct((E, 3 * FEAT), jnp.float32),
                  jax.ShapeDtypeStruct((E, 3 * FEAT), jnp.float32)),
        scratch_types=[
            pltpu.VMEM((_GB,), jnp.int32),
            pltpu.VMEM((_GB,), jnp.int32),
            pltpu.VMEM((_GB, 4 * FEAT), jnp.float32),
            pltpu.VMEM((_GB, 3 * FEAT), jnp.float32),
            pltpu.VMEM((_GB, 3 * FEAT), jnp.float32),
            pltpu.SemaphoreType.DMA,
            pltpu.SemaphoreType.DMA,
            pltpu.SemaphoreType.DMA,
        ],
    )
    return f(phi_all, vt2d, dst_i32, src_i32)


@jax.jit
def kernel(s_j, v_j, r_ij, nbrs, cg_adj, W1, b1, W2, b2, Wd, bd):
    h = s_j @ W1 + b1
    h = h * jax.nn.sigmoid(h)
    phi_all = h @ W2 + b2

    src = nbrs[:, 0]
    dst = nbrs[:, 1]
    vt = jnp.swapaxes(v_j, 1, 2)
    vt2d = vt.reshape(N, 3 * FEAT)
    phi_e, vde, vse = _sc_gather(phi_all, vt2d, dst.astype(jnp.int32),
                                 src.astype(jnp.int32))
    adj = cg_adj[src, dst][:, None]

    msgs = _edge_pass(r_ij, phi_e, vde, vse, adj, Wd, bd[None, :])
    acc = jnp.zeros((N, 4 * FEAT), jnp.float32).at[src].add(msgs)
    ds_i = acc[:, :FEAT]
    dv_i = jnp.stack([acc[:, FEAT:2 * FEAT], acc[:, 2 * FEAT:3 * FEAT],
                      acc[:, 3 * FEAT:]], axis=-1)
    return ds_i, dv_i


# speedup vs baseline: 9.5603x; 1.2259x over previous
import functools

import jax
import jax.numpy as jnp
from jax import lax
from jax.experimental import pallas as pl
from jax.experimental.pallas import tpu as pltpu
from jax.experimental.pallas import tpu_sc as plsc

N = 10000
E = 320000
FEAT = 128
NRBF = 20
CUTOFF = 5.0
EPS = 1e-15

BLK_E = 1600


def _edge_block_kernel(r_ref, phi_ref, vd_ref, vs_ref, adj_ref, wd_ref,
                       bd_ref, coef_ref, ms_ref):
    r = r_ref[...]
    rx = r[:, 0:1]
    ry = r[:, 1:2]
    rz = r[:, 2:3]
    d2 = rx * rx + ry * ry + rz * rz + EPS
    dist = jnp.sqrt(d2)
    inv_d = 1.0 / dist
    ux, uy, uz = rx * inv_d, ry * inv_d, rz * inv_d

    rbf = jnp.sin(coef_ref[...] * dist) * inv_d
    rbf_feats = jnp.dot(rbf, wd_ref[...],
                        preferred_element_type=jnp.float32) + bd_ref[...]
    env = jnp.where(dist < CUTOFF,
                    0.5 * (jnp.cos((jnp.pi / CUTOFF) * dist) + 1.0), 0.0)
    w_s = rbf_feats * env

    phi = phi_ref[...]
    inv_out = phi * w_s
    adj = adj_ref[...]
    s0 = inv_out[:, 0 * FEAT:1 * FEAT] * adj
    s1 = inv_out[:, 1 * FEAT:2 * FEAT] * adj
    s2 = inv_out[:, 2 * FEAT:3 * FEAT] * adj
    s3 = inv_out[:, 3 * FEAT:4 * FEAT] * adj

    vd = vd_ref[...]
    vs = vs_ref[...]
    vdx, vdy, vdz = (vd[:, 0:FEAT], vd[:, FEAT:2 * FEAT],
                     vd[:, 2 * FEAT:3 * FEAT])
    vsx, vsy, vsz = (vs[:, 0:FEAT], vs[:, FEAT:2 * FEAT],
                     vs[:, 2 * FEAT:3 * FEAT])
    cx = vsy * vdz - vsz * vdy
    cy = vsz * vdx - vsx * vdz
    cz = vsx * vdy - vsy * vdx

    ms_ref[:, 0 * FEAT:1 * FEAT] = s1
    ms_ref[:, 1 * FEAT:2 * FEAT] = s2 * ux + s0 * vdx + s3 * cx
    ms_ref[:, 2 * FEAT:3 * FEAT] = s2 * uy + s0 * vdy + s3 * cy
    ms_ref[:, 3 * FEAT:4 * FEAT] = s2 * uz + s0 * vdz + s3 * cz


def _edge_pass(r_ij, phi_e, vde, vse, adj, Wd4, bd4, interpret=False):
    ne = r_ij.shape[0]
    grid = (ne // BLK_E,)
    eb = lambda w: pl.BlockSpec((BLK_E, w), lambda i: (i, 0))
    full = lambda a, b: pl.BlockSpec((a, b), lambda i: (0, 0))
    coef = (jnp.arange(1, NRBF + 1, dtype=jnp.float32)
            * (jnp.pi / CUTOFF))[None, :]
    return pl.pallas_call(
        _edge_block_kernel,
        grid=grid,
        in_specs=[eb(3), eb(4 * FEAT), eb(3 * FEAT), eb(3 * FEAT),
                  eb(1), full(NRBF, 4 * FEAT), full(1, 4 * FEAT),
                  full(1, NRBF)],
        out_specs=eb(4 * FEAT),
        out_shape=jax.ShapeDtypeStruct((ne, 4 * FEAT), jnp.float32),
        interpret=interpret,
    )(r_ij, phi_e, vde, vse, adj, Wd4, bd4, coef)


_NC, _NS = 2, 16
_NW = _NC * _NS
_NSLAB = 5
_ES = E // _NSLAB
_EPW = _ES // _NW
_GB = 80


def _sc_gather_body(phi_hbm, vt_hbm, dst_hbm, src_hbm,
                    phi_out, vd_out, vs_out,
                    idx_d, idx_s, phi_rows, vd_rows, vs_rows,
                    sem_p, sem_d, sem_s):
    wid = lax.axis_index("s") * _NC + lax.axis_index("c")
    wbase = wid * _EPW

    def chunk(i, _):
        base = wbase + i * _GB
        pltpu.sync_copy(dst_hbm.at[pl.ds(base, _GB)], idx_d)
        pltpu.sync_copy(src_hbm.at[pl.ds(base, _GB)], idx_s)
        cp_p = pltpu.async_copy(phi_hbm.at[idx_d], phi_rows, sem_p)
        cp_d = pltpu.async_copy(vt_hbm.at[idx_d], vd_rows, sem_d)
        cp_s = pltpu.async_copy(vt_hbm.at[idx_s], vs_rows, sem_s)
        cp_p.wait()
        pltpu.sync_copy(phi_rows, phi_out.at[pl.ds(base, _GB)])
        cp_d.wait()
        pltpu.sync_copy(vd_rows, vd_out.at[pl.ds(base, _GB)])
        cp_s.wait()
        pltpu.sync_copy(vs_rows, vs_out.at[pl.ds(base, _GB)])
        return _

    lax.fori_loop(0, _EPW // _GB, chunk, 0)


def _sc_gather(phi_all, vt2d, dst_i32, src_i32):
    mesh = plsc.VectorSubcoreMesh(core_axis_name="c", subcore_axis_name="s")
    f = pl.kernel(
        _sc_gather_body,
        mesh=mesh,
        out_type=(jax.ShapeDtypeStruct((_ES, 4 * FEAT), jnp.float32),
                  jax.ShapeDtypeStruct((_ES, 3 * FEAT), jnp.float32),
                  jax.ShapeDtypeStruct((_ES, 3 * FEAT), jnp.float32)),
        scratch_types=[
            pltpu.VMEM((_GB,), jnp.int32),
            pltpu.VMEM((_GB,), jnp.int32),
            pltpu.VMEM((_GB, 4 * FEAT), jnp.float32),
            pltpu.VMEM((_GB, 3 * FEAT), jnp.float32),
            pltpu.VMEM((_GB, 3 * FEAT), jnp.float32),
            pltpu.SemaphoreType.DMA,
            pltpu.SemaphoreType.DMA,
            pltpu.SemaphoreType.DMA,
        ],
    )
    return f(phi_all, vt2d, dst_i32, src_i32)


@jax.jit
def kernel(s_j, v_j, r_ij, nbrs, cg_adj, W1, b1, W2, b2, Wd, bd):
    h = s_j @ W1 + b1
    h = h * jax.nn.sigmoid(h)
    phi_all = h @ W2 + b2

    src = nbrs[:, 0].astype(jnp.int32)
    dst = nbrs[:, 1].astype(jnp.int32)
    vt = jnp.swapaxes(v_j, 1, 2)
    vt2d = vt.reshape(N, 3 * FEAT)
    adj = cg_adj[src, dst][:, None]

    bd2 = bd[None, :]
    acc = jnp.zeros((N, 4 * FEAT), jnp.float32)
    for k in range(_NSLAB):
        sl = slice(k * _ES, (k + 1) * _ES)
        phi_e, vde, vse = _sc_gather(phi_all, vt2d, dst[sl], src[sl])
        msgs = _edge_pass(r_ij[sl], phi_e, vde, vse, adj[sl], Wd, bd2)
        acc = acc.at[src[sl]].add(msgs)

    ds_i = acc[:, :FEAT]
    dv_i = jnp.stack([acc[:, FEAT:2 * FEAT], acc[:, 2 * FEAT:3 * FEAT],
                      acc[:, 3 * FEAT:]], axis=-1)
    return ds_i, dv_i


# speedup vs baseline: 10.6629x; 1.1153x over previous
import functools

import jax
import jax.numpy as jnp
from jax import lax
from jax.experimental import pallas as pl
from jax.experimental.pallas import tpu as pltpu
from jax.experimental.pallas import tpu_sc as plsc

N = 10000
E = 320000
FEAT = 128
NRBF = 20
CUTOFF = 5.0
EPS = 1e-15

BLK_E = 1600


def _unpack_lo(x):
    return lax.bitcast_convert_type(x << 16, jnp.float32)


def _unpack_hi(x):
    return lax.bitcast_convert_type(x & -65536, jnp.float32)


def _edge_block_kernel(r_ref, phi_ref, vd_ref, vs_ref, adj_ref, wd_ref,
                       bd_ref, coef_ref, ms_ref):
    r = r_ref[...]
    rx = r[:, 0:1]
    ry = r[:, 1:2]
    rz = r[:, 2:3]
    d2 = rx * rx + ry * ry + rz * rz + EPS
    dist = jnp.sqrt(d2)
    inv_d = 1.0 / dist
    ux, uy, uz = rx * inv_d, ry * inv_d, rz * inv_d

    rbf = jnp.sin(coef_ref[...] * dist) * inv_d
    rbf_feats = jnp.dot(rbf, wd_ref[...],
                        preferred_element_type=jnp.float32) + bd_ref[...]
    env = jnp.where(dist < CUTOFF,
                    0.5 * (jnp.cos((jnp.pi / CUTOFF) * dist) + 1.0), 0.0)
    w_s = rbf_feats * env

    phi_p = phi_ref[...]
    phi_lo = _unpack_lo(phi_p)
    phi_hi = _unpack_hi(phi_p)
    adj = adj_ref[...]
    wlo = w_s[:, 0:2 * FEAT]
    whi = w_s[:, 2 * FEAT:4 * FEAT]
    ilo = phi_lo * wlo
    ihi = phi_hi * whi
    s0 = ilo[:, 0:FEAT] * adj
    s1 = ilo[:, FEAT:2 * FEAT] * adj
    s2 = ihi[:, 0:FEAT] * adj
    s3 = ihi[:, FEAT:2 * FEAT] * adj

    vd_p = vd_ref[...]
    vs_p = vs_ref[...]
    vd_lo = _unpack_lo(vd_p)
    vd_hi = _unpack_hi(vd_p)
    vs_lo = _unpack_lo(vs_p)
    vs_hi = _unpack_hi(vs_p)
    vdx, vdy, vdz = vd_lo[:, 0:FEAT], vd_lo[:, FEAT:2 * FEAT], vd_hi[:, 0:FEAT]
    vsx, vsy, vsz = vs_lo[:, 0:FEAT], vs_lo[:, FEAT:2 * FEAT], vs_hi[:, 0:FEAT]
    cx = vsy * vdz - vsz * vdy
    cy = vsz * vdx - vsx * vdz
    cz = vsx * vdy - vsy * vdx

    ms_ref[:, 0 * FEAT:1 * FEAT] = s1
    ms_ref[:, 1 * FEAT:2 * FEAT] = s2 * ux + s0 * vdx + s3 * cx
    ms_ref[:, 2 * FEAT:3 * FEAT] = s2 * uy + s0 * vdy + s3 * cy
    ms_ref[:, 3 * FEAT:4 * FEAT] = s2 * uz + s0 * vdz + s3 * cz


def _edge_pass(r_ij, phi_e, vde, vse, adj, Wd4, bd4, interpret=False):
    ne = r_ij.shape[0]
    grid = (ne // BLK_E,)
    eb = lambda w: pl.BlockSpec((BLK_E, w), lambda i: (i, 0))
    full = lambda a, b: pl.BlockSpec((a, b), lambda i: (0, 0))
    coef = (jnp.arange(1, NRBF + 1, dtype=jnp.float32)
            * (jnp.pi / CUTOFF))[None, :]
    return pl.pallas_call(
        _edge_block_kernel,
        grid=grid,
        in_specs=[eb(3), eb(2 * FEAT), eb(2 * FEAT), eb(2 * FEAT),
                  eb(1), full(NRBF, 4 * FEAT), full(1, 4 * FEAT),
                  full(1, NRBF)],
        out_specs=eb(4 * FEAT),
        out_shape=jax.ShapeDtypeStruct((ne, 4 * FEAT), jnp.float32),
        interpret=interpret,
    )(r_ij, phi_e, vde, vse, adj, Wd4, bd4, coef)


_NC, _NS = 2, 16
_NW = _NC * _NS
_NSLAB = 5
_ES = E // _NSLAB
_EPW = _ES // _NW
_GB = 80


def _sc_gather_body(phi_hbm, vt_hbm, dst_hbm, src_hbm,
                    phi_out, vd_out, vs_out,
                    idx_d, idx_s, phi_rows, vd_rows, vs_rows,
                    sem_p, sem_d, sem_s):
    wid = lax.axis_index("s") * _NC + lax.axis_index("c")
    wbase = wid * _EPW

    def chunk(i, _):
        base = wbase + i * _GB
        pltpu.sync_copy(dst_hbm.at[pl.ds(base, _GB)], idx_d)
        pltpu.sync_copy(src_hbm.at[pl.ds(base, _GB)], idx_s)
        cp_p = pltpu.async_copy(phi_hbm.at[idx_d], phi_rows, sem_p)
        cp_d = pltpu.async_copy(vt_hbm.at[idx_d], vd_rows, sem_d)
        cp_s = pltpu.async_copy(vt_hbm.at[idx_s], vs_rows, sem_s)
        cp_p.wait()
        pltpu.sync_copy(phi_rows, phi_out.at[pl.ds(base, _GB)])
        cp_d.wait()
        pltpu.sync_copy(vd_rows, vd_out.at[pl.ds(base, _GB)])
        cp_s.wait()
        pltpu.sync_copy(vs_rows, vs_out.at[pl.ds(base, _GB)])
        return _

    lax.fori_loop(0, _EPW // _GB, chunk, 0)


def _sc_gather(phi_all, vt2d, dst_i32, src_i32):
    mesh = plsc.VectorSubcoreMesh(core_axis_name="c", subcore_axis_name="s")
    f = pl.kernel(
        _sc_gather_body,
        mesh=mesh,
        out_type=(jax.ShapeDtypeStruct((_ES, 2 * FEAT), jnp.int32),
                  jax.ShapeDtypeStruct((_ES, 2 * FEAT), jnp.int32),
                  jax.ShapeDtypeStruct((_ES, 2 * FEAT), jnp.int32)),
        scratch_types=[
            pltpu.VMEM((_GB,), jnp.int32),
            pltpu.VMEM((_GB,), jnp.int32),
            pltpu.VMEM((_GB, 2 * FEAT), jnp.int32),
            pltpu.VMEM((_GB, 2 * FEAT), jnp.int32),
            pltpu.VMEM((_GB, 2 * FEAT), jnp.int32),
            pltpu.SemaphoreType.DMA,
            pltpu.SemaphoreType.DMA,
            pltpu.SemaphoreType.DMA,
        ],
    )
    return f(phi_all, vt2d, dst_i32, src_i32)


@jax.jit
def kernel(s_j, v_j, r_ij, nbrs, cg_adj, W1, b1, W2, b2, Wd, bd):
    h = s_j @ W1 + b1
    h = h * jax.nn.sigmoid(h)
    phi_all = h @ W2 + b2

    src = nbrs[:, 0].astype(jnp.int32)
    dst = nbrs[:, 1].astype(jnp.int32)
    vt = jnp.swapaxes(v_j, 1, 2)
    vt2d = vt.reshape(N, 3 * FEAT)
    adj = cg_adj[src, dst][:, None]

    def pack(x):
        xb = x.astype(jnp.bfloat16)
        lo = lax.bitcast_convert_type(xb[:, :2 * FEAT], jnp.uint16)
        hi = lax.bitcast_convert_type(xb[:, 2 * FEAT:], jnp.uint16)
        packed = (hi.astype(jnp.uint32) << 16) | lo.astype(jnp.uint32)
        return lax.bitcast_convert_type(packed, jnp.int32)

    phi_p = pack(phi_all)
    vt_p = pack(jnp.pad(vt2d, ((0, 0), (0, FEAT))))

    bd2 = bd[None, :]
    acc = jnp.zeros((N, 4 * FEAT), jnp.float32)
    for k in range(_NSLAB):
        sl = slice(k * _ES, (k + 1) * _ES)
        phi_e, vde, vse = _sc_gather(phi_p, vt_p, dst[sl], src[sl])
        msgs = _edge_pass(r_ij[sl], phi_e, vde, vse, adj[sl], Wd, bd2)
        acc = acc.at[src[sl]].add(msgs)

    ds_i = acc[:, :FEAT]
    dv_i = jnp.stack([acc[:, FEAT:2 * FEAT], acc[:, 2 * FEAT:3 * FEAT],
                      acc[:, 3 * FEAT:]], axis=-1)
    return ds_i, dv_i
